# baseline (speedup 1.0000x reference)
import functools

import jax
import jax.numpy as jnp
from jax import lax
from jax.experimental import pallas as pl
from jax.experimental.pallas import tpu as pltpu
from jax.experimental.pallas import tpu_sc as plsc

NC = 2
NS = 16
L = 16
NW = NC * NS

BATCH = 16384
SEQ = 48
HALF = 24
D = 64
RW = BATCH // NW
IW = RW * SEQ
G = 16
CROWS = G * SEQ
NCHUNK = RW // G
NSTR = 2
RPS = G // NSTR
SROWS = RPS * SEQ


_mesh = plsc.VectorSubcoreMesh(
    core_axis_name="c", subcore_axis_name="s", num_cores=NC, num_subcores=NS
)

_row_buf = pltpu.VMEM((SROWS, D), jnp.bfloat16)


@functools.partial(
    pl.kernel,
    out_type=jax.ShapeDtypeStruct((NW, L), jnp.float32),
    mesh=_mesh,
    compiler_params=pltpu.CompilerParams(
        use_tc_tiling_on_sc=False, needs_layout_passes=False),
    scratch_types=(
        [pltpu.VMEM((IW,), jnp.int32)]
        + [_row_buf] * (2 * NSTR)
        + [pltpu.VMEM((L,), jnp.float32)]
        + [pltpu.SemaphoreType.DMA] * (2 * NSTR)
    ),
)
def _pair_loss(table_hbm, xf_hbm, out_hbm, xi_v, *rest):
    bufs = [rest[:NSTR], rest[NSTR:2 * NSTR]]
    acc_v = rest[2 * NSTR]
    sems = [rest[2 * NSTR + 1:2 * NSTR + 1 + NSTR],
            rest[2 * NSTR + 1 + NSTR:2 * NSTR + 1 + 2 * NSTR]]

    wid = lax.axis_index("s") * NC + lax.axis_index("c")
    base = wid * IW

    pltpu.sync_copy(xf_hbm.at[pl.ds(base, IW)], xi_v)

    def start(g, slot):
        for t in range(NSTR):
            idx = pl.ds(g * CROWS + t * SROWS, SROWS)
            pltpu.async_copy(table_hbm.at[xi_v.at[idx]], bufs[slot][t],
                             sems[slot][t])

    def drain(slot):
        for t in range(NSTR):
            pltpu.make_async_copy(table_hbm.at[pl.ds(0, SROWS)],
                                  bufs[slot][t], sems[slot][t]).wait()

    def compute(slot, accs):
        for t in range(NSTR):
            e_buf = bufs[slot][t]

            def brow(r, accs, e_buf=e_buf):
                out = list(accs)
                rb = r * SEQ
                for s in range(HALF):
                    for j in range(2):
                        av = e_buf[rb + s, pl.ds(j * 2 * L, 2 * L)]
                        bv = e_buf[rb + s + HALF, pl.ds(j * 2 * L, 2 * L)]
                        dv = av - bv
                        d0, d1 = plsc.unpack(
                            dv, format=plsc.PackFormat.INTERLEAVED,
                            preferred_element_type=jnp.float32)
                        out[2 * j] = out[2 * j] + d0 * d0
                        out[2 * j + 1] = out[2 * j + 1] + d1 * d1
                return tuple(out)

            accs = lax.fori_loop(0, RPS, brow, accs)
        return accs

    zeros = jnp.zeros((L,), jnp.float32)
    accs = (zeros, zeros, zeros, zeros)

    start(0, 0)

    def body(h, accs):
        g = 2 * h
        start(g + 1, 1)
        drain(0)
        accs = compute(0, accs)
        start(g + 2, 0)
        drain(1)
        return compute(1, accs)

    accs = lax.fori_loop(0, NCHUNK // 2 - 1, body, accs)

    start(NCHUNK - 1, 1)
    drain(0)
    accs = compute(0, accs)
    drain(1)
    accs = compute(1, accs)

    acc_v[...] = (accs[0] + accs[1]) + (accs[2] + accs[3])
    pltpu.sync_copy(acc_v, out_hbm.at[wid])


def kernel(x, embd_size, table):
    partials = _pair_loss(table.astype(jnp.bfloat16), x.reshape(-1))
    return jnp.sum(partials)


# speedup vs baseline: 1.7017x; 1.7017x over previous
import functools

import jax
import jax.numpy as jnp
from jax import lax
from jax.experimental import pallas as pl
from jax.experimental.pallas import tpu as pltpu
from jax.experimental.pallas import tpu_sc as plsc

NC = 2
NS = 16
L = 16
NW = NC * NS

BATCH = 16384
SEQ = 48
HALF = 24
D = 64
RW = BATCH // NW
G = 16
CROWS = G * SEQ
NCHUNK = RW // G


_mesh = plsc.VectorSubcoreMesh(
    core_axis_name="c", subcore_axis_name="s", num_cores=NC, num_subcores=NS
)


@functools.partial(
    pl.kernel,
    out_type=jax.ShapeDtypeStruct((NW, L), jnp.float32),
    mesh=_mesh,
    compiler_params=pltpu.CompilerParams(
        use_tc_tiling_on_sc=False, needs_layout_passes=False),
    scratch_types=[
        pltpu.VMEM((RW, SEQ), jnp.int32),
        pltpu.VMEM((CROWS, D), jnp.float32),
        pltpu.VMEM((CROWS, D), jnp.float32),
        pltpu.VMEM((L,), jnp.float32),
        pltpu.SemaphoreType.DMA,
        pltpu.SemaphoreType.DMA,
    ],
)
def _pair_loss(table_hbm, x_hbm, out_hbm, x2_v, e0, e1, acc_v, s0, s1):
    bufs = (e0, e1)
    sems = (s0, s1)

    wid = lax.axis_index("s") * NC + lax.axis_index("c")

    pltpu.sync_copy(x_hbm.at[pl.ds(wid * RW, RW)], x2_v)

    def start(g, slot):
        for t in range(G):
            pltpu.async_copy(table_hbm.at[x2_v.at[g * G + t]],
                             bufs[slot].at[pl.ds(t * SEQ, SEQ)], sems[slot])

    def drain(slot):
        for t in range(G):
            pltpu.make_async_copy(table_hbm.at[pl.ds(0, SEQ)],
                                  bufs[slot].at[pl.ds(t * SEQ, SEQ)],
                                  sems[slot]).wait()

    def compute(slot, accs):
        e_buf = bufs[slot]

        def brow(r, accs):
            rb = r * SEQ

            def pair(s, accs):
                out = list(accs)
                for j in range(4):
                    av = e_buf[rb + s, pl.ds(j * L, L)]
                    bv = e_buf[rb + s + HALF, pl.ds(j * L, L)]
                    dv = av - bv
                    out[j] = out[j] + dv * dv
                return tuple(out)

            return lax.fori_loop(0, HALF, pair, accs, unroll=4)

        return lax.fori_loop(0, G, brow, accs)

    zeros = jnp.zeros((L,), jnp.float32)
    accs = (zeros, zeros, zeros, zeros)

    start(0, 0)

    def body(h, accs):
        g = 2 * h
        start(g + 1, 1)
        drain(0)
        accs = compute(0, accs)
        start(g + 2, 0)
        drain(1)
        return compute(1, accs)

    accs = lax.fori_loop(0, NCHUNK // 2 - 1, body, accs)

    start(NCHUNK - 1, 1)
    drain(0)
    accs = compute(0, accs)
    drain(1)
    accs = compute(1, accs)

    acc_v[...] = (accs[0] + accs[1]) + (accs[2] + accs[3])
    pltpu.sync_copy(acc_v, out_hbm.at[wid])


def kernel(x, embd_size, table):
    partials = _pair_loss(table, x)
    return jnp.sum(partials)
